# baseline (speedup 1.0000x reference)
import functools

import jax
import jax.numpy as jnp
from jax import lax
from jax.experimental import pallas as pl
from jax.experimental.pallas import tpu as pltpu
from jax.experimental.pallas import tpu_sc as plsc

N_NODES = 10000
N_EDGES = 320000
D_FEAT = 128
HIDDEN = 3
N_CLASSES = 10

NC = 2
NS = 16
NW = NC * NS
VL = 16
N_PAD = 10240
EPW = N_EDGES // NW
EPW_2VL = (EPW // (2 * VL)) * (2 * VL)
NP3 = HIDDEN * N_PAD
PAYW_T = 4

_MESH = plsc.VectorSubcoreMesh(core_axis_name="c", subcore_axis_name="s")
_SC_PARAMS = pltpu.CompilerParams(use_tc_tiling_on_sc=False,
                                  needs_layout_passes=False)


@jax.jit
def _sc_histogram(ei):
    @functools.partial(
        pl.kernel,
        out_type=jax.ShapeDtypeStruct((NW * N_PAD,), jnp.float32),
        mesh=_MESH,
        compiler_params=_SC_PARAMS,
        scratch_types=[
            pltpu.VMEM((EPW,), jnp.int32),
            pltpu.VMEM((N_PAD,), jnp.float32),
            pltpu.SemaphoreType.DMA,
        ],
    )
    def histo(ei_hbm, out_hbm, idx_v, deg_v, sem):
        c = lax.axis_index("c")
        s = lax.axis_index("s")
        wid = s * NC + c
        cp = pltpu.async_copy(
            ei_hbm.at[pl.ds(N_EDGES + wid * EPW, EPW)], idx_v, sem)

        @pl.loop(0, N_PAD, step=4 * VL)
        def _(i):
            for u in range(4):
                deg_v[pl.ds(i + u * VL, VL)] = jnp.zeros((VL,), jnp.float32)

        cp.wait()

        ones = jnp.ones((VL,), jnp.float32)

        @pl.loop(0, EPW_2VL, step=2 * VL)
        def _(i):
            d16a = idx_v[pl.ds(i, VL)]
            d16b = idx_v[pl.ds(i + VL, VL)]
            plsc.addupdate_scatter(deg_v, [d16a], ones)
            plsc.addupdate_scatter(deg_v, [d16b], ones)

        @pl.loop(EPW_2VL, EPW, step=VL)
        def _(i):
            plsc.addupdate_scatter(deg_v, [idx_v[pl.ds(i, VL)]], ones)

        pltpu.sync_copy(deg_v, out_hbm.at[pl.ds(wid * N_PAD, N_PAD)])

    return histo(ei)


@jax.jit
def _sc_gather_scatter(ei, yflat):
    @functools.partial(
        pl.kernel,
        out_type=jax.ShapeDtypeStruct((HIDDEN * NW * N_PAD,), jnp.float32),
        mesh=_MESH,
        compiler_params=_SC_PARAMS,
        scratch_types=[
            pltpu.VMEM((EPW,), jnp.int32),
            pltpu.VMEM((EPW,), jnp.int32),
            pltpu.VMEM((NP3,), jnp.float32),
            pltpu.VMEM((NP3,), jnp.float32),
            pltpu.VMEM_SHARED((NP3,), jnp.float32),
            pltpu.SemaphoreType.DMA,
            pltpu.SemaphoreType.DMA,
        ],
    )
    def gscat(ei_hbm, y_hbm, out_hbm, si_v, di_v, y_v, acc_v, y_sh,
              sem1, sem2):
        c = lax.axis_index("c")
        s = lax.axis_index("s")
        wid = s * NC + c

        @pl.when(s == 0)
        def _():
            pltpu.sync_copy(y_hbm, y_sh)

        cp1 = pltpu.async_copy(ei_hbm.at[pl.ds(wid * EPW, EPW)], si_v, sem1)
        cp2 = pltpu.async_copy(
            ei_hbm.at[pl.ds(N_EDGES + wid * EPW, EPW)], di_v, sem2)

        @pl.loop(0, NP3, step=4 * VL)
        def _(i):
            for u in range(4):
                acc_v[pl.ds(i + u * VL, VL)] = jnp.zeros((VL,), jnp.float32)

        cp1.wait()
        cp2.wait()
        plsc.subcore_barrier()
        pltpu.sync_copy(y_sh, y_v)

        @pl.loop(0, EPW_2VL, step=2 * VL)
        def _(i):
            s16a = si_v[pl.ds(i, VL)]
            d16a = di_v[pl.ds(i, VL)]
            s16b = si_v[pl.ds(i + VL, VL)]
            d16b = di_v[pl.ds(i + VL, VL)]
            for k in range(HIDDEN):
                va = plsc.load_gather(y_v, [s16a + (k * N_PAD)])
                plsc.addupdate_scatter(acc_v, [d16a + (k * N_PAD)], va)
            for k in range(HIDDEN):
                vb = plsc.load_gather(y_v, [s16b + (k * N_PAD)])
                plsc.addupdate_scatter(acc_v, [d16b + (k * N_PAD)], vb)

        @pl.loop(EPW_2VL, EPW, step=VL)
        def _(i):
            s16 = si_v[pl.ds(i, VL)]
            d16 = di_v[pl.ds(i, VL)]
            for k in range(HIDDEN):
                v = plsc.load_gather(y_v, [s16 + (k * N_PAD)])
                plsc.addupdate_scatter(acc_v, [d16 + (k * N_PAD)], v)

        for k in range(HIDDEN):
            pltpu.sync_copy(
                acc_v.at[pl.ds(k * N_PAD, N_PAD)],
                out_hbm.at[pl.ds((k * NW + wid) * N_PAD, N_PAD)])

    return gscat(ei, yflat)


def _tc_xw(x_pad, W4):
    def body(x_ref, w_ref, xw_ref):
        xw_ref[...] = lax.dot_general(
            w_ref[...], x_ref[...],
            dimension_numbers=(((0,), (1,)), ((), ())),
            preferred_element_type=jnp.float32)

    return pl.pallas_call(
        body,
        out_shape=jax.ShapeDtypeStruct((PAYW_T, N_PAD), jnp.float32),
    )(x_pad, W4)


def _tc_norm(xwT, degp2):
    def body(xw_ref, degp_ref, y_ref, dis_ref):
        deg = 1.0 + jnp.sum(degp_ref[...], axis=0, keepdims=True)
        dis = lax.rsqrt(deg)
        y_ref[...] = xw_ref[...] * dis
        dis_ref[...] = dis

    return pl.pallas_call(
        body,
        out_shape=[
            jax.ShapeDtypeStruct((PAYW_T, N_PAD), jnp.float32),
            jax.ShapeDtypeStruct((1, N_PAD), jnp.float32),
        ],
    )(xwT, degp2)


def _tc_final(accp, yT, disT, bgT, W_out, boT):
    def body(accp_ref, y_ref, dis_ref, bg_ref, wo_ref, bo_ref, h_ref, z_ref):
        parts = [
            jnp.sum(accp_ref[pl.ds(k * NW, NW), :], axis=0, keepdims=True)
            for k in range(HIDDEN)
        ]
        acc = jnp.concatenate(parts, axis=0)
        agg = dis_ref[...] * (acc + y_ref[pl.ds(0, HIDDEN), :])
        h = jnp.maximum(agg + bg_ref[...], 0.0)
        zT = lax.dot_general(
            wo_ref[...], h,
            dimension_numbers=(((0,), (0,)), ((), ())),
            preferred_element_type=jnp.float32) + bo_ref[...]
        h_ref[...] = jnp.transpose(h)[:N_NODES]
        z_ref[...] = jnp.transpose(zT)[:N_NODES]

    return pl.pallas_call(
        body,
        out_shape=[
            jax.ShapeDtypeStruct((N_NODES, HIDDEN), jnp.float32),
            jax.ShapeDtypeStruct((N_NODES, N_CLASSES), jnp.float32),
        ],
    )(accp, yT, disT, bgT, W_out, boT)


def kernel(x, edge_index, W_gcn, b_gcn, W_out, b_out):
    ei = edge_index.astype(jnp.int32).reshape(2 * N_EDGES)
    x_pad = jnp.pad(x, ((0, N_PAD - N_NODES), (0, 0)))
    W4 = jnp.pad(W_gcn, ((0, 0), (0, PAYW_T - HIDDEN)))
    bgT = b_gcn.reshape(HIDDEN, 1)
    boT = b_out.reshape(N_CLASSES, 1)

    degp = _sc_histogram(ei)
    xwT = _tc_xw(x_pad, W4)
    degp2 = degp.reshape(NW, N_PAD)
    yT, disT = _tc_norm(xwT, degp2)
    yflat = yT[:HIDDEN].reshape(NP3)
    accp = _sc_gather_scatter(ei, yflat)
    accp2 = accp.reshape(HIDDEN * NW, N_PAD)
    h, z = _tc_final(accp2, yT, disT, bgT, W_out, boT)

    return h, z


# speedup vs baseline: 1.0987x; 1.0987x over previous
import functools

import jax
import jax.numpy as jnp
from jax import lax
from jax.experimental import pallas as pl
from jax.experimental.pallas import tpu as pltpu
from jax.experimental.pallas import tpu_sc as plsc

N_NODES = 10000
N_EDGES = 320000
D_FEAT = 128
HIDDEN = 3
N_CLASSES = 10

NC = 2
NS = 16
NW = NC * NS
VL = 16
N_PAD = 10240
EPW = N_EDGES // NW
EPW_2VL = (EPW // (2 * VL)) * (2 * VL)
NP3 = HIDDEN * N_PAD
PAYW_T = 4

_MESH = plsc.VectorSubcoreMesh(core_axis_name="c", subcore_axis_name="s")
_SC_PARAMS = pltpu.CompilerParams(use_tc_tiling_on_sc=False,
                                  needs_layout_passes=False)


@jax.jit
def _sc_histogram(ei):
    @functools.partial(
        pl.kernel,
        out_type=jax.ShapeDtypeStruct((NW * N_PAD,), jnp.float32),
        mesh=_MESH,
        compiler_params=_SC_PARAMS,
        scratch_types=[
            pltpu.VMEM((EPW,), jnp.int32),
            pltpu.VMEM((N_PAD,), jnp.float32),
            pltpu.SemaphoreType.DMA,
        ],
    )
    def histo(ei_hbm, out_hbm, idx_v, deg_v, sem):
        c = lax.axis_index("c")
        s = lax.axis_index("s")
        wid = s * NC + c
        cp = pltpu.async_copy(
            ei_hbm.at[pl.ds(N_EDGES + wid * EPW, EPW)], idx_v, sem)

        @pl.loop(0, N_PAD, step=4 * VL)
        def _(i):
            for u in range(4):
                deg_v[pl.ds(i + u * VL, VL)] = jnp.zeros((VL,), jnp.float32)

        cp.wait()

        ones = jnp.ones((VL,), jnp.float32)

        @pl.loop(0, EPW_2VL, step=2 * VL)
        def _(i):
            d16a = idx_v[pl.ds(i, VL)]
            d16b = idx_v[pl.ds(i + VL, VL)]
            plsc.addupdate_scatter(deg_v, [d16a], ones)
            plsc.addupdate_scatter(deg_v, [d16b], ones)

        @pl.loop(EPW_2VL, EPW, step=VL)
        def _(i):
            plsc.addupdate_scatter(deg_v, [idx_v[pl.ds(i, VL)]], ones)

        pltpu.sync_copy(deg_v, out_hbm.at[pl.ds(wid * N_PAD, N_PAD)])

    return histo(ei)


@jax.jit
def _sc_gather_scatter(ei, yflat):
    @functools.partial(
        pl.kernel,
        out_type=jax.ShapeDtypeStruct((HIDDEN * NW * N_PAD,), jnp.float32),
        mesh=_MESH,
        compiler_params=_SC_PARAMS,
        scratch_types=[
            pltpu.VMEM((EPW,), jnp.int32),
            pltpu.VMEM((EPW,), jnp.int32),
            pltpu.VMEM((NP3,), jnp.float32),
            pltpu.VMEM((NP3,), jnp.float32),
            pltpu.VMEM_SHARED((NP3,), jnp.float32),
            pltpu.SemaphoreType.DMA,
            pltpu.SemaphoreType.DMA,
        ],
    )
    def gscat(ei_hbm, y_hbm, out_hbm, si_v, di_v, y_v, acc_v, y_sh,
              sem1, sem2):
        c = lax.axis_index("c")
        s = lax.axis_index("s")
        wid = s * NC + c

        @pl.when(s == 0)
        def _():
            pltpu.sync_copy(y_hbm, y_sh)

        cp1 = pltpu.async_copy(ei_hbm.at[pl.ds(wid * EPW, EPW)], si_v, sem1)
        cp2 = pltpu.async_copy(
            ei_hbm.at[pl.ds(N_EDGES + wid * EPW, EPW)], di_v, sem2)

        @pl.loop(0, NP3, step=4 * VL)
        def _(i):
            for u in range(4):
                acc_v[pl.ds(i + u * VL, VL)] = jnp.zeros((VL,), jnp.float32)

        cp1.wait()
        cp2.wait()
        plsc.subcore_barrier()
        pltpu.sync_copy(y_sh, y_v)

        @pl.loop(0, EPW_2VL, step=2 * VL)
        def _(i):
            s16a = si_v[pl.ds(i, VL)]
            d16a = di_v[pl.ds(i, VL)]
            s16b = si_v[pl.ds(i + VL, VL)]
            d16b = di_v[pl.ds(i + VL, VL)]
            for k in range(HIDDEN):
                va = plsc.load_gather(y_v, [s16a + (k * N_PAD)])
                plsc.addupdate_scatter(acc_v, [d16a + (k * N_PAD)], va)
            for k in range(HIDDEN):
                vb = plsc.load_gather(y_v, [s16b + (k * N_PAD)])
                plsc.addupdate_scatter(acc_v, [d16b + (k * N_PAD)], vb)

        @pl.loop(EPW_2VL, EPW, step=VL)
        def _(i):
            s16 = si_v[pl.ds(i, VL)]
            d16 = di_v[pl.ds(i, VL)]
            for k in range(HIDDEN):
                v = plsc.load_gather(y_v, [s16 + (k * N_PAD)])
                plsc.addupdate_scatter(acc_v, [d16 + (k * N_PAD)], v)

        for k in range(HIDDEN):
            pltpu.sync_copy(
                acc_v.at[pl.ds(k * N_PAD, N_PAD)],
                out_hbm.at[pl.ds((k * NW + wid) * N_PAD, N_PAD)])

    return gscat(ei, yflat)


def _tc_xw(x_pad, W4):
    def body(x_ref, w_ref, xw_ref):
        xw_ref[...] = lax.dot_general(
            w_ref[...], x_ref[...],
            dimension_numbers=(((0,), (1,)), ((), ())),
            preferred_element_type=jnp.float32)

    return pl.pallas_call(
        body,
        out_shape=jax.ShapeDtypeStruct((PAYW_T, N_PAD), jnp.float32),
    )(x_pad, W4)


def _tc_norm(xwT, degp2):
    def body(xw_ref, degp_ref, y_ref, dis_ref):
        deg = 1.0 + jnp.sum(degp_ref[...], axis=0, keepdims=True)
        dis = lax.rsqrt(deg)
        y_ref[...] = xw_ref[...] * dis
        dis_ref[...] = dis

    return pl.pallas_call(
        body,
        out_shape=[
            jax.ShapeDtypeStruct((PAYW_T, N_PAD), jnp.float32),
            jax.ShapeDtypeStruct((1, N_PAD), jnp.float32),
        ],
    )(xwT, degp2)


def _tc_final(accp, yT, disT, bgT, W_out, boT):
    def body(accp_ref, y_ref, dis_ref, bg_ref, wo_ref, bo_ref, h_ref, z_ref):
        parts = [
            jnp.sum(accp_ref[pl.ds(k * NW, NW), :], axis=0, keepdims=True)
            for k in range(HIDDEN)
        ]
        acc = jnp.concatenate(parts, axis=0)
        agg = dis_ref[...] * (acc + y_ref[pl.ds(0, HIDDEN), :])
        h = jnp.maximum(agg + bg_ref[...], 0.0)
        h_ref[...] = h
        z_ref[...] = lax.dot_general(
            wo_ref[...], h,
            dimension_numbers=(((0,), (0,)), ((), ())),
            preferred_element_type=jnp.float32) + bo_ref[...]

    return pl.pallas_call(
        body,
        out_shape=[
            jax.ShapeDtypeStruct((HIDDEN, N_PAD), jnp.float32),
            jax.ShapeDtypeStruct((N_CLASSES, N_PAD), jnp.float32),
        ],
    )(accp, yT, disT, bgT, W_out, boT)


def kernel(x, edge_index, W_gcn, b_gcn, W_out, b_out):
    ei = edge_index.astype(jnp.int32).reshape(2 * N_EDGES)
    x_pad = jnp.pad(x, ((0, N_PAD - N_NODES), (0, 0)))
    W4 = jnp.pad(W_gcn, ((0, 0), (0, PAYW_T - HIDDEN)))
    bgT = b_gcn.reshape(HIDDEN, 1)
    boT = b_out.reshape(N_CLASSES, 1)

    degp = _sc_histogram(ei)
    xwT = _tc_xw(x_pad, W4)
    degp2 = degp.reshape(NW, N_PAD)
    yT, disT = _tc_norm(xwT, degp2)
    yflat = yT[:HIDDEN].reshape(NP3)
    accp = _sc_gather_scatter(ei, yflat)
    accp2 = accp.reshape(HIDDEN * NW, N_PAD)
    hT, zT = _tc_final(accp2, yT, disT, bgT, W_out, boT)

    return hT[:, :N_NODES].T, zT[:, :N_NODES].T
